# speedup vs baseline: 10.4389x; 10.4389x over previous
import functools

import jax
import jax.numpy as jnp
from jax.experimental import pallas as pl

BLOCK = 512
K = 32
BISECT_ITERS = 30


def _body(x_ref, w1_ref, b1_ref, w2_ref, b2_ref, o_ref):
    x = x_ref[...]
    z = jnp.dot(x, w1_ref[...], preferred_element_type=jnp.float32)
    z = z + b1_ref[...]
    code = 1.0 / (1.0 + jnp.exp(-z))

    ikey = jax.lax.bitcast_convert_type(code, jnp.int32)
    rows = ikey.shape[0]
    lo0 = jnp.zeros((rows, 1), jnp.int32)
    hi0 = jnp.full((rows, 1), 0x3F800001, jnp.int32)

    def step(_, carry):
        lo, hi = carry
        mid = lo + jax.lax.shift_right_logical(hi - lo, 1)
        cnt = jnp.sum((ikey >= mid).astype(jnp.int32), axis=1, keepdims=True)
        ge = cnt >= K
        lo = jnp.where(ge, mid, lo)
        hi = jnp.where(ge, hi, mid)
        return lo, hi

    lo, _ = jax.lax.fori_loop(0, BISECT_ITERS, step, (lo0, hi0))

    masked = jnp.where(ikey >= lo, code, 0.0)
    y = jnp.dot(masked, w2_ref[...], preferred_element_type=jnp.float32)
    y = y + b2_ref[...]
    o_ref[...] = 1.0 / (1.0 + jnp.exp(-y))


@jax.jit
def kernel(input, W1, b1, W2, b2):
    batch, in_dim = input.shape
    code_dim = W1.shape[1]
    grid = (batch // BLOCK,)
    return pl.pallas_call(
        _body,
        grid=grid,
        in_specs=[
            pl.BlockSpec((BLOCK, in_dim), lambda i: (i, 0)),
            pl.BlockSpec((in_dim, code_dim), lambda i: (0, 0)),
            pl.BlockSpec((1, code_dim), lambda i: (0, 0)),
            pl.BlockSpec((code_dim, in_dim), lambda i: (0, 0)),
            pl.BlockSpec((1, in_dim), lambda i: (0, 0)),
        ],
        out_specs=pl.BlockSpec((BLOCK, in_dim), lambda i: (i, 0)),
        out_shape=jax.ShapeDtypeStruct((batch, in_dim), jnp.float32),
    )(input, W1, b1.reshape(1, -1), W2, b2.reshape(1, -1))
